# baseline (speedup 1.0000x reference)
import jax
import jax.numpy as jnp
from jax import lax
from jax.experimental import pallas as pl
from jax.experimental.pallas import tpu as pltpu
from jax.experimental.pallas import tpu_sc as plsc

NC, NS, L = 2, 16, 16
NW = NC * NS
B = 4096
L1 = 20
D = 64
KD = D // L
BPW = B // NW
C = 8
NCH = BPW // C
RPC = C * L1
SPC = RPC // 8


def _body(xi, yi, xf, yf, hp, vp, out,
          idx0, idx1, yidx0, yidx1, rows0, rows1, xv0, xv1,
          vrow0, vrow1, yv0, yv1, outv, pbuf0, pbuf1,
          semg0, semg1, semv0, semv1, semi0, semi1):
    idx_v = (idx0, idx1)
    pbuf = (pbuf0, pbuf1)
    yidx_v = (yidx0, yidx1)
    rows_v = (rows0, rows1)
    x_v = (xv0, xv1)
    vrow_v = (vrow0, vrow1)
    y_v = (yv0, yv1)
    semg = (semg0, semg1)
    semv = (semv0, semv1)
    semi = (semi0, semi1)

    wid = lax.axis_index("s") * NC + lax.axis_index("c")
    rbase0 = wid * (BPW * L1)
    bbase0 = wid * BPW
    lane_iota = lax.iota(jnp.int32, L)

    def prep_idx(ch, s):
        rb = rbase0 + ch * RPC
        bb = bbase0 + ch * C
        pltpu.make_async_copy(xi.at[pl.ds(rb, RPC)],
                              idx_v[s].at[pl.ds(0, RPC)], semi[s]).start()
        pltpu.make_async_copy(yi.at[pl.ds(bb, C)],
                              yidx_v[s].at[pl.ds(0, C)], semi[s]).start()

    def prep(ch, s):
        rb = rbase0 + ch * RPC
        bb = bbase0 + ch * C
        pltpu.make_async_copy(xi.at[pl.ds(0, RPC)],
                              idx_v[s].at[pl.ds(0, RPC)], semi[s]).wait()
        pltpu.make_async_copy(yi.at[pl.ds(0, C)],
                              yidx_v[s].at[pl.ds(0, C)], semi[s]).wait()

        for t in range(RPC // L):
            v = idx_v[s][pl.ds(t * L, L)]
            pbuf[s][pl.ds(t * L, L)] = v & 3

        def hrow(j, carry):
            r = idx_v[s][pl.ds(j, L)][0]
            g = lax.shift_right_logical(r, 2)
            q = lax.shift_right_logical(g, 3)
            tr = g & 7
            jq = lax.shift_right_logical(j, 3)
            jr = j & 7
            pltpu.make_async_copy(hp.at[pl.ds(q, 1), pl.ds(tr, 1), :],
                                  rows_v[s].at[pl.ds(jq, 1), pl.ds(jr, 1), :],
                                  semg[s]).start()
            return carry

        lax.fori_loop(0, RPC, hrow, 0)

        def vrow(j, carry):
            r = yidx_v[s][pl.ds(j, L)][0]
            q = lax.shift_right_logical(r, 3)
            tr = r & 7
            pltpu.make_async_copy(vp.at[pl.ds(q, 1), pl.ds(tr, 1), :],
                                  vrow_v[s].at[pl.ds(0, 1), pl.ds(j, 1), :],
                                  semv[s]).start()
            return carry

        lax.fori_loop(0, C, vrow, 0)

        pltpu.make_async_copy(xf.at[pl.ds(rb // 8, SPC), :, :], x_v[s],
                              semg[s]).start()
        pltpu.make_async_copy(yf.at[pl.ds(bb // 8, 1), :, :], y_v[s],
                              semv[s]).start()

    def compute(ch, s):
        pltpu.make_async_copy(hp.at[pl.ds(0, SPC), :, :],
                              rows_v[s], semg[s]).wait()
        pb = pbuf[s]
        pltpu.make_async_copy(xf.at[pl.ds(0, SPC), :, :], x_v[s],
                              semg[s]).wait()
        pltpu.make_async_copy(vp.at[pl.ds(0, 1), :, :],
                              vrow_v[s], semv[s]).wait()
        pltpu.make_async_copy(yf.at[pl.ds(0, 1), :, :], y_v[s],
                              semv[s]).wait()
        rows = rows_v[s]
        xv = x_v[s]
        vrow = vrow_v[s]
        yv = y_v[s]

        def ebody(e, carry):
            rb = e * L1
            acc = [jnp.zeros((L,), jnp.float32) for _ in range(KD)]
            for l in range(L1):
                j = rb + l
                jq = lax.shift_right_logical(j, 3)
                jr = j & 7
                p = pb[pl.ds(j, L)][0]
                for kk in range(KD // 2):
                    hw = rows[jq, jr, pl.ds(p * 32 + kk * L, L)]
                    hbf = plsc.bitcast(hw, jnp.bfloat16)
                    ha, hb_ = plsc.unpack(
                        hbf, format=plsc.PackFormat.INTERLEAVED)
                    xa = xv[jq, jr, pl.ds(kk * 2 * L, L)]
                    xb = xv[jq, jr, pl.ds(kk * 2 * L + L, L)]
                    acc[2 * kk] = acc[2 * kk] + ha * xa
                    acc[2 * kk + 1] = acc[2 * kk + 1] + hb_ * xb
            t = jnp.zeros((L,), jnp.float32)
            for k in range(KD):
                w = vrow[0, e, pl.ds(k * L, L)] * yv[0, e, pl.ds(k * L, L)]
                t = t + acc[k] * w
            s_ = jnp.sum(t)
            return jnp.where(lane_iota == e, s_, carry)

        tot = lax.fori_loop(0, C, ebody, jnp.zeros((L,), jnp.float32))
        outv[:] = tot
        bb = bbase0 + ch * C
        pltpu.sync_copy(outv.at[pl.ds(0, C)], out.at[pl.ds(bb, C)])

    prep_idx(0, 0)
    prep_idx(1, 1)
    prep(0, 0)
    prep(1, 1)

    def pair(c2, carry):
        ch0 = c2 * 2
        prep_idx(ch0 + 2, 0)
        compute(ch0, 0)
        prep(ch0 + 2, 0)
        prep_idx(ch0 + 3, 1)
        compute(ch0 + 1, 1)
        prep(ch0 + 3, 1)
        return carry

    lax.fori_loop(0, NCH // 2 - 1, pair, 0)
    compute(NCH - 2, 0)
    compute(NCH - 1, 1)


@jax.jit
def _gmf_sc(xi, yi, xf, yf, hp, vp):
    mesh = plsc.VectorSubcoreMesh(core_axis_name="c", subcore_axis_name="s")
    scratch = [
        pltpu.VMEM((RPC + L,), jnp.int32), pltpu.VMEM((RPC + L,), jnp.int32),
        pltpu.VMEM((C + L,), jnp.int32), pltpu.VMEM((C + L,), jnp.int32),
        pltpu.VMEM((SPC, 8, 128), jnp.int32), pltpu.VMEM((SPC, 8, 128), jnp.int32),
        pltpu.VMEM((SPC, 8, D), jnp.float32), pltpu.VMEM((SPC, 8, D), jnp.float32),
        pltpu.VMEM((1, 8, D), jnp.float32), pltpu.VMEM((1, 8, D), jnp.float32),
        pltpu.VMEM((1, 8, D), jnp.float32), pltpu.VMEM((1, 8, D), jnp.float32),
        pltpu.VMEM((L,), jnp.float32),
        pltpu.VMEM((RPC + L,), jnp.int32), pltpu.VMEM((RPC + L,), jnp.int32),
        pltpu.SemaphoreType.DMA, pltpu.SemaphoreType.DMA,
        pltpu.SemaphoreType.DMA, pltpu.SemaphoreType.DMA,
        pltpu.SemaphoreType.DMA, pltpu.SemaphoreType.DMA,
    ]
    run = pl.kernel(
        _body,
        out_type=jax.ShapeDtypeStruct((B,), jnp.float32),
        mesh=mesh,
        scratch_types=scratch,
        compiler_params=pltpu.CompilerParams(
            needs_layout_passes=False, use_tc_tiling_on_sc=True),
    )
    return run(xi, yi, xf, yf, hp, vp)


_COL_ORDER = tuple(
    blk + off
    for blk in (0, 32)
    for i in range(16)
    for off in (i, 16 + i)
)


def kernel(x_idx, y_idx, x, y, human_table, virus_table):
    xi = x_idx.reshape(B * L1)
    yi = y_idx.reshape(B)
    xf = x.reshape(B * L1 // 8, 8, D)
    yf = y.reshape(B // 8, 8, D)
    hb = human_table[:, jnp.array(_COL_ORDER)].astype(jnp.bfloat16)
    hpk = lax.bitcast_convert_type(hb.reshape(250000, 128, 2), jnp.int32)
    hp = hpk.reshape(31250, 8, 128)
    vp = virus_table.reshape(100000 // 8, 8, D)
    out = _gmf_sc(xi, yi, xf, yf, hp, vp)
    return out.reshape(B, 1)


# speedup vs baseline: 7.3122x; 7.3122x over previous
import jax
import jax.numpy as jnp
from jax import lax
from jax.experimental import pallas as pl
from jax.experimental.pallas import tpu as pltpu
from jax.experimental.pallas import tpu_sc as plsc

NC, NS, L = 2, 16, 16
NW = NC * NS
B = 4096
L1 = 20
D = 64
KD = D // L
BPW = B // NW
C = 8
NCH = BPW // C
RPC = C * L1
SPC = RPC // 8


def _body(xi, yi, xf, yf, hp, vp, out,
          idx0, idx1, yidx0, yidx1, rows0, rows1, xv0, xv1,
          vrow0, vrow1, yv0, yv1, outv, pbuf0, pbuf1,
          semg0, semg1, semv0, semv1, semi0, semi1):
    idx_v = (idx0, idx1)
    pbuf = (pbuf0, pbuf1)
    yidx_v = (yidx0, yidx1)
    rows_v = (rows0, rows1)
    x_v = (xv0, xv1)
    vrow_v = (vrow0, vrow1)
    y_v = (yv0, yv1)
    semg = (semg0, semg1)
    semv = (semv0, semv1)
    semi = (semi0, semi1)

    wid = lax.axis_index("s") * NC + lax.axis_index("c")
    rbase0 = wid * (BPW * L1)
    bbase0 = wid * BPW
    lane_iota = lax.iota(jnp.int32, L)

    def prep_idx(ch, s):
        rb = rbase0 + ch * RPC
        bb = bbase0 + ch * C
        pltpu.make_async_copy(xi.at[pl.ds(rb, RPC)],
                              idx_v[s].at[pl.ds(0, RPC)], semi[s]).start()
        pltpu.make_async_copy(yi.at[pl.ds(bb, C)],
                              yidx_v[s].at[pl.ds(0, C)], semi[s]).start()

    def prep(ch, s):
        rb = rbase0 + ch * RPC
        bb = bbase0 + ch * C
        pltpu.make_async_copy(xi.at[pl.ds(0, RPC)],
                              idx_v[s].at[pl.ds(0, RPC)], semi[s]).wait()
        pltpu.make_async_copy(yi.at[pl.ds(0, C)],
                              yidx_v[s].at[pl.ds(0, C)], semi[s]).wait()

        for t in range(RPC // L):
            v = idx_v[s][pl.ds(t * L, L)]
            pbuf[s][pl.ds(t * L, L)] = v & 3

        def hrow(j, carry):
            r = idx_v[s][pl.ds(j, L)][0]
            g = lax.shift_right_logical(r, 2)
            q = lax.shift_right_logical(g, 3)
            tr = g & 7
            jq = lax.shift_right_logical(j, 3)
            jr = j & 7
            pltpu.make_async_copy(hp.at[pl.ds(q, 1), pl.ds(tr, 1), :],
                                  rows_v[s].at[pl.ds(jq, 1), pl.ds(jr, 1), :],
                                  semg[s]).start()
            return carry

        lax.fori_loop(0, RPC, hrow, 0)

        def vrow(j, carry):
            r = yidx_v[s][pl.ds(j, L)][0]
            q = lax.shift_right_logical(r, 3)
            tr = r & 7
            pltpu.make_async_copy(vp.at[pl.ds(q, 1), pl.ds(tr, 1), :],
                                  vrow_v[s].at[pl.ds(0, 1), pl.ds(j, 1), :],
                                  semv[s]).start()
            return carry

        lax.fori_loop(0, C, vrow, 0)

        pltpu.make_async_copy(xf.at[pl.ds(rb // 8, SPC), :, :], x_v[s],
                              semg[s]).start()
        pltpu.make_async_copy(yf.at[pl.ds(bb // 8, 1), :, :], y_v[s],
                              semv[s]).start()

    def compute(ch, s):
        pltpu.make_async_copy(hp.at[pl.ds(0, SPC), :, :],
                              rows_v[s], semg[s]).wait()
        pb = pbuf[s]
        pltpu.make_async_copy(xf.at[pl.ds(0, SPC), :, :], x_v[s],
                              semg[s]).wait()
        pltpu.make_async_copy(vp.at[pl.ds(0, 1), :, :],
                              vrow_v[s], semv[s]).wait()
        pltpu.make_async_copy(yf.at[pl.ds(0, 1), :, :], y_v[s],
                              semv[s]).wait()
        rows = rows_v[s]
        xv = x_v[s]
        vrow = vrow_v[s]
        yv = y_v[s]

        def ebody(e, carry):
            rb = e * L1
            acc = [jnp.zeros((L,), jnp.float32) for _ in range(KD)]
            for l in range(L1):
                j = rb + l
                jq = lax.shift_right_logical(j, 3)
                jr = j & 7
                p = pb[pl.ds(j, L)][0]
                for kk in range(KD // 2):
                    hw = rows[jq, jr, pl.ds(p * 32 + kk * L, L)]
                    hbf = plsc.bitcast(hw, jnp.bfloat16)
                    ha, hb_ = plsc.unpack(
                        hbf, format=plsc.PackFormat.INTERLEAVED)
                    xa = xv[jq, jr, pl.ds(kk * 2 * L, L)]
                    xb = xv[jq, jr, pl.ds(kk * 2 * L + L, L)]
                    acc[2 * kk] = acc[2 * kk] + ha * xa
                    acc[2 * kk + 1] = acc[2 * kk + 1] + hb_ * xb
            t = jnp.zeros((L,), jnp.float32)
            for k in range(KD):
                w = vrow[0, e, pl.ds(k * L, L)] * yv[0, e, pl.ds(k * L, L)]
                t = t + acc[k] * w
            s_ = jnp.sum(t)
            return jnp.where(lane_iota == e, s_, carry)

        tot = lax.fori_loop(0, C, ebody, jnp.zeros((L,), jnp.float32))
        outv[:] = tot
        bb = bbase0 + ch * C
        pltpu.sync_copy(outv.at[pl.ds(0, C)], out.at[pl.ds(bb, C)])

    prep_idx(0, 0)
    prep_idx(1, 1)
    prep(0, 0)
    prep(1, 1)

    def pair(c2, carry):
        ch0 = c2 * 2
        prep_idx(ch0 + 2, 0)
        compute(ch0, 0)
        prep(ch0 + 2, 0)
        prep_idx(ch0 + 3, 1)
        compute(ch0 + 1, 1)
        prep(ch0 + 3, 1)
        return carry

    lax.fori_loop(0, NCH // 2 - 1, pair, 0)
    compute(NCH - 2, 0)
    compute(NCH - 1, 1)


@jax.jit
def _gmf_sc(xi, yi, xf, yf, hp, vp):
    mesh = plsc.VectorSubcoreMesh(core_axis_name="c", subcore_axis_name="s")
    scratch = [
        pltpu.VMEM((RPC + L,), jnp.int32), pltpu.VMEM((RPC + L,), jnp.int32),
        pltpu.VMEM((C + L,), jnp.int32), pltpu.VMEM((C + L,), jnp.int32),
        pltpu.VMEM((SPC, 8, 128), jnp.int32), pltpu.VMEM((SPC, 8, 128), jnp.int32),
        pltpu.VMEM((SPC, 8, D), jnp.float32), pltpu.VMEM((SPC, 8, D), jnp.float32),
        pltpu.VMEM((1, 8, D), jnp.float32), pltpu.VMEM((1, 8, D), jnp.float32),
        pltpu.VMEM((1, 8, D), jnp.float32), pltpu.VMEM((1, 8, D), jnp.float32),
        pltpu.VMEM((L,), jnp.float32),
        pltpu.VMEM((RPC + L,), jnp.int32), pltpu.VMEM((RPC + L,), jnp.int32),
        pltpu.SemaphoreType.DMA, pltpu.SemaphoreType.DMA,
        pltpu.SemaphoreType.DMA, pltpu.SemaphoreType.DMA,
        pltpu.SemaphoreType.DMA, pltpu.SemaphoreType.DMA,
    ]
    run = pl.kernel(
        _body,
        out_type=jax.ShapeDtypeStruct((B,), jnp.float32),
        mesh=mesh,
        scratch_types=scratch,
        compiler_params=pltpu.CompilerParams(
            needs_layout_passes=False, use_tc_tiling_on_sc=True),
    )
    return run(xi, yi, xf, yf, hp, vp)


_COL_ORDER = tuple(
    blk + off
    for blk in (0, 32)
    for i in range(16)
    for off in (i, 16 + i)
)


def kernel(x_idx, y_idx, x, y, human_table, virus_table):
    xi = x_idx.reshape(B * L1)
    yi = y_idx.reshape(B)
    xf = x.reshape(B * L1 // 8, 8, D)
    yf = y.reshape(B // 8, 8, D)
    hb = jnp.transpose(
        human_table.reshape(1000000, 2, 2, L), (0, 1, 3, 2)
    ).astype(jnp.bfloat16)
    hpk = lax.bitcast_convert_type(hb.reshape(250000, 128, 2), jnp.int32)
    hp = hpk.reshape(31250, 8, 128)
    vp = virus_table.reshape(100000 // 8, 8, D)
    out = _gmf_sc(xi, yi, xf, yf, hp, vp)
    return out.reshape(B, 1)


# speedup vs baseline: 66.9265x; 9.1527x over previous
import jax
import jax.numpy as jnp
from jax import lax
from jax.experimental import pallas as pl
from jax.experimental.pallas import tpu as pltpu
from jax.experimental.pallas import tpu_sc as plsc

NC, NS, L = 2, 16, 16
NW = NC * NS
B = 4096
L1 = 20
D = 64
KD = D // L
BPW = B // NW
C = 8
NCH = BPW // C
RPC = C * L1
SPC = RPC // 8


def _body(xi, yi, xf, yf, hp, vp, out,
          idx0, idx1, yidx0, yidx1, rows0, rows1, xv0, xv1,
          vrow0, vrow1, yv0, yv1, outv,
          semg0, semg1, semv0, semv1, semi0, semi1):
    idx_v = (idx0, idx1)
    yidx_v = (yidx0, yidx1)
    rows_v = (rows0, rows1)
    x_v = (xv0, xv1)
    vrow_v = (vrow0, vrow1)
    y_v = (yv0, yv1)
    semg = (semg0, semg1)
    semv = (semv0, semv1)
    semi = (semi0, semi1)

    wid = lax.axis_index("s") * NC + lax.axis_index("c")
    rbase0 = wid * (BPW * L1)
    bbase0 = wid * BPW
    lane_iota = lax.iota(jnp.int32, L)

    def prep_idx(ch, s):
        rb = rbase0 + ch * RPC
        bb = bbase0 + ch * C
        pltpu.make_async_copy(xi.at[pl.ds(rb, RPC)],
                              idx_v[s].at[pl.ds(0, RPC)], semi[s]).start()
        pltpu.make_async_copy(yi.at[pl.ds(bb, C)],
                              yidx_v[s].at[pl.ds(0, C)], semi[s]).start()

    def prep(ch, s):
        rb = rbase0 + ch * RPC
        bb = bbase0 + ch * C
        pltpu.make_async_copy(xi.at[pl.ds(0, RPC)],
                              idx_v[s].at[pl.ds(0, RPC)], semi[s]).wait()
        pltpu.make_async_copy(yi.at[pl.ds(0, C)],
                              yidx_v[s].at[pl.ds(0, C)], semi[s]).wait()

        def hrow(t, carry):
            j0 = t * L
            rv = idx_v[s][pl.ds(j0, L)]
            qv = lax.shift_right_logical(rv, 3)
            sv = rv & 7
            for u in range(L):
                j = j0 + u
                jq = lax.shift_right_logical(j, 3)
                jr = j & 7
                pltpu.make_async_copy(
                    hp.at[pl.ds(qv[u], 1), pl.ds(sv[u], 1), :],
                    rows_v[s].at[pl.ds(jq, 1), pl.ds(jr, 1), :],
                    semg[s]).start()
            return carry

        lax.fori_loop(0, RPC // L, hrow, 0)

        def vrow(j, carry):
            r = yidx_v[s][pl.ds(j, L)][0]
            q = lax.shift_right_logical(r, 3)
            tr = r & 7
            pltpu.make_async_copy(vp.at[pl.ds(q, 1), pl.ds(tr, 1), :],
                                  vrow_v[s].at[pl.ds(0, 1), pl.ds(j, 1), :],
                                  semv[s]).start()
            return carry

        lax.fori_loop(0, C, vrow, 0)

        pltpu.make_async_copy(xf.at[pl.ds(rb // 8, SPC), :, :], x_v[s],
                              semg[s]).start()
        pltpu.make_async_copy(yf.at[pl.ds(bb // 8, 1), :, :], y_v[s],
                              semv[s]).start()

    def compute(ch, s):
        pltpu.make_async_copy(hp.at[pl.ds(0, SPC), :, :],
                              rows_v[s], semg[s]).wait()
        pltpu.make_async_copy(xf.at[pl.ds(0, SPC), :, :], x_v[s],
                              semg[s]).wait()
        pltpu.make_async_copy(vp.at[pl.ds(0, 1), :, :],
                              vrow_v[s], semv[s]).wait()
        pltpu.make_async_copy(yf.at[pl.ds(0, 1), :, :], y_v[s],
                              semv[s]).wait()
        rows = rows_v[s]
        xv = x_v[s]
        vrow = vrow_v[s]
        yv = y_v[s]

        def ebody(e, carry):
            rb = e * L1
            acc = [jnp.zeros((L,), jnp.float32) for _ in range(KD)]
            for l in range(L1):
                j = rb + l
                jq = lax.shift_right_logical(j, 3)
                jr = j & 7
                for k in range(KD):
                    h = rows[jq, jr, pl.ds(k * L, L)]
                    xx = xv[jq, jr, pl.ds(k * L, L)]
                    acc[k] = acc[k] + h * xx
            t = jnp.zeros((L,), jnp.float32)
            for k in range(KD):
                w = vrow[0, e, pl.ds(k * L, L)] * yv[0, e, pl.ds(k * L, L)]
                t = t + acc[k] * w
            s_ = jnp.sum(t)
            return jnp.where(lane_iota == e, s_, carry)

        tot = lax.fori_loop(0, C, ebody, jnp.zeros((L,), jnp.float32))
        outv[:] = tot
        bb = bbase0 + ch * C
        pltpu.sync_copy(outv.at[pl.ds(0, C)], out.at[pl.ds(bb, C)])

    prep_idx(0, 0)
    prep_idx(1, 1)
    prep(0, 0)
    prep(1, 1)

    def pair(c2, carry):
        ch0 = c2 * 2
        prep_idx(ch0 + 2, 0)
        compute(ch0, 0)
        prep(ch0 + 2, 0)
        prep_idx(ch0 + 3, 1)
        compute(ch0 + 1, 1)
        prep(ch0 + 3, 1)
        return carry

    lax.fori_loop(0, NCH // 2 - 1, pair, 0)
    compute(NCH - 2, 0)
    compute(NCH - 1, 1)


@jax.jit
def _gmf_sc(xi, yi, xf, yf, hp, vp):
    mesh = plsc.VectorSubcoreMesh(core_axis_name="c", subcore_axis_name="s")
    scratch = [
        pltpu.VMEM((RPC + L,), jnp.int32), pltpu.VMEM((RPC + L,), jnp.int32),
        pltpu.VMEM((C + L,), jnp.int32), pltpu.VMEM((C + L,), jnp.int32),
        pltpu.VMEM((SPC, 8, D), jnp.float32), pltpu.VMEM((SPC, 8, D), jnp.float32),
        pltpu.VMEM((SPC, 8, D), jnp.float32), pltpu.VMEM((SPC, 8, D), jnp.float32),
        pltpu.VMEM((1, 8, D), jnp.float32), pltpu.VMEM((1, 8, D), jnp.float32),
        pltpu.VMEM((1, 8, D), jnp.float32), pltpu.VMEM((1, 8, D), jnp.float32),
        pltpu.VMEM((L,), jnp.float32),
        pltpu.SemaphoreType.DMA, pltpu.SemaphoreType.DMA,
        pltpu.SemaphoreType.DMA, pltpu.SemaphoreType.DMA,
        pltpu.SemaphoreType.DMA, pltpu.SemaphoreType.DMA,
    ]
    run = pl.kernel(
        _body,
        out_type=jax.ShapeDtypeStruct((B,), jnp.float32),
        mesh=mesh,
        scratch_types=scratch,
        compiler_params=pltpu.CompilerParams(
            needs_layout_passes=False, use_tc_tiling_on_sc=True),
    )
    return run(xi, yi, xf, yf, hp, vp)


def kernel(x_idx, y_idx, x, y, human_table, virus_table):
    xi = x_idx.reshape(B * L1)
    yi = y_idx.reshape(B)
    xf = x.reshape(B * L1 // 8, 8, D)
    yf = y.reshape(B // 8, 8, D)
    hp = human_table.reshape(1000000 // 8, 8, D)
    vp = virus_table.reshape(100000 // 8, 8, D)
    out = _gmf_sc(xi, yi, xf, yf, hp, vp)
    return out.reshape(B, 1)
